# baseline (speedup 1.0000x reference)
import jax
import jax.numpy as jnp
from jax import lax
from jax.experimental import pallas as pl

V = 1024
C = 64


def _prep_kernel(cb_ref, cb2_ref, csq_ref):
    cb = cb_ref[...]
    cb2_ref[...] = -2.0 * cb
    csq_ref[...] = jnp.sum(cb * cb, axis=1, keepdims=True)


def _vq_kernel(x_ref, cb_ref, cb2_ref, csq_ref, xd_ref, counts_ref):
    i = pl.program_id(0)

    nb = x_ref.shape[0]
    xb = jnp.concatenate([x_ref[b] for b in range(nb)], axis=1)
    cb = cb_ref[...]
    W = xb.shape[1]

    s = jnp.dot(cb2_ref[...], xb) + csq_ref[...]

    minval = jnp.min(s, axis=0, keepdims=True)
    iota_col = lax.broadcasted_iota(jnp.int32, (V, 1), 0).astype(jnp.float32)
    idx = jnp.min(jnp.where(s <= minval, iota_col, float(V)),
                  axis=0, keepdims=True)
    onehot = jnp.where(iota_col == idx, 1.0, 0.0)

    xd = lax.dot_general(cb, onehot, (((0,), (0,)), ((), ())))
    T = W // nb
    for b in range(nb):
        xd_ref[b] = xd[:, b * T:(b + 1) * T]

    part = jnp.sum(onehot, axis=1, keepdims=True)
    prev = jnp.where(i == 0, 0.0, counts_ref[...])
    counts_ref[...] = prev + part


def _perp_kernel(counts_ref, perp_ref):
    counts = counts_ref[...]
    prob = counts / jnp.sum(counts)
    ent = jnp.sum(prob * jnp.log(prob + 1e-07),
                  axis=0, keepdims=True)
    perp_ref[...] = jnp.exp(-ent)


def kernel(x, codebook):
    N, width, T = x.shape
    cb2, csq = pl.pallas_call(
        _prep_kernel,
        out_shape=[
            jax.ShapeDtypeStruct((V, C), jnp.float32),
            jax.ShapeDtypeStruct((V, 1), jnp.float32),
        ],
    )(codebook)
    NB = 4
    xd, counts = pl.pallas_call(
        _vq_kernel,
        grid=(N // NB,),
        in_specs=[
            pl.BlockSpec((NB, width, T), lambda i: (i, 0, 0)),
            pl.BlockSpec((V, C), lambda i: (0, 0)),
            pl.BlockSpec((V, C), lambda i: (0, 0)),
            pl.BlockSpec((V, 1), lambda i: (0, 0)),
        ],
        out_specs=[
            pl.BlockSpec((NB, width, T), lambda i: (i, 0, 0)),
            pl.BlockSpec((V, 1), lambda i: (0, 0)),
        ],
        out_shape=[
            jax.ShapeDtypeStruct((N, width, T), jnp.float32),
            jax.ShapeDtypeStruct((V, 1), jnp.float32),
        ],
    )(x, codebook, cb2, csq)
    perp = pl.pallas_call(
        _perp_kernel,
        out_shape=jax.ShapeDtypeStruct((1, 1), jnp.float32),
    )(counts)
    return (xd, perp[0, 0])


# speedup vs baseline: 1.0256x; 1.0256x over previous
import jax
import jax.numpy as jnp
from jax import lax
from jax.experimental import pallas as pl

V = 1024
C = 64


def _prep_kernel(cb_ref, cb2_ref, csq_ref):
    cb = cb_ref[...]
    cb2_ref[...] = -2.0 * cb
    csq_ref[...] = jnp.sum(cb * cb, axis=1, keepdims=True)


def _vq_kernel(x_ref, cb_ref, cb2_ref, csq_ref, xd_ref, counts_ref):
    i = pl.program_id(0)

    nb = x_ref.shape[0]
    xb = jnp.concatenate([x_ref[b] for b in range(nb)], axis=1)
    cb = cb_ref[...]
    W = xb.shape[1]

    s = jnp.dot(cb2_ref[...].astype(jnp.bfloat16), xb.astype(jnp.bfloat16),
                preferred_element_type=jnp.float32) + csq_ref[...]

    minval = jnp.min(s, axis=0, keepdims=True)
    iota_col = lax.broadcasted_iota(jnp.int32, (V, 1), 0).astype(jnp.float32)
    idx = jnp.min(jnp.where(s <= minval, iota_col, float(V)),
                  axis=0, keepdims=True)
    onehot = jnp.where(iota_col == idx, 1.0, 0.0)

    xd = lax.dot_general(cb, onehot, (((0,), (0,)), ((), ())))
    T = W // nb
    for b in range(nb):
        xd_ref[b] = xd[:, b * T:(b + 1) * T]

    part = jnp.sum(onehot, axis=1, keepdims=True)
    prev = jnp.where(i == 0, 0.0, counts_ref[...])
    counts_ref[...] = prev + part


def _perp_kernel(counts_ref, perp_ref):
    counts = counts_ref[...]
    prob = counts / jnp.sum(counts)
    ent = jnp.sum(prob * jnp.log(prob + 1e-07),
                  axis=0, keepdims=True)
    perp_ref[...] = jnp.exp(-ent)


def kernel(x, codebook):
    N, width, T = x.shape
    cb2, csq = pl.pallas_call(
        _prep_kernel,
        out_shape=[
            jax.ShapeDtypeStruct((V, C), jnp.float32),
            jax.ShapeDtypeStruct((V, 1), jnp.float32),
        ],
    )(codebook)
    NB = 4
    xd, counts = pl.pallas_call(
        _vq_kernel,
        grid=(N // NB,),
        in_specs=[
            pl.BlockSpec((NB, width, T), lambda i: (i, 0, 0)),
            pl.BlockSpec((V, C), lambda i: (0, 0)),
            pl.BlockSpec((V, C), lambda i: (0, 0)),
            pl.BlockSpec((V, 1), lambda i: (0, 0)),
        ],
        out_specs=[
            pl.BlockSpec((NB, width, T), lambda i: (i, 0, 0)),
            pl.BlockSpec((V, 1), lambda i: (0, 0)),
        ],
        out_shape=[
            jax.ShapeDtypeStruct((N, width, T), jnp.float32),
            jax.ShapeDtypeStruct((V, 1), jnp.float32),
        ],
    )(x, codebook, cb2, csq)
    perp = pl.pallas_call(
        _perp_kernel,
        out_shape=jax.ShapeDtypeStruct((1, 1), jnp.float32),
    )(counts)
    return (xd, perp[0, 0])
